# baseline (speedup 1.0000x reference)
import functools

import jax
import jax.numpy as jnp
from jax import lax
from jax.experimental import pallas as pl
from jax.experimental.pallas import tpu as pltpu
from jax.experimental.pallas import tpu_sc as plsc

_N = 4194304
_NCOMP = 8
_NW = 32
_BLK = 8192
_N_SC = 786432
_N_TC = _N - _N_SC
_PER_W = _N_SC // _NW
_NBLK = _PER_W // _BLK

_TC_BLK = 131072
_TC_CHUNK = 4096

_LN2 = 0.6931471805599453
_SQRT2 = 1.4142135623730951


def _log2_sc(s):
    bits = lax.bitcast_convert_type(s, jnp.int32)
    e = lax.shift_right_arithmetic(bits, 23) - 127
    mbits = (bits & jnp.int32(0x007FFFFF)) | jnp.int32(0x3F800000)
    mf = lax.bitcast_convert_type(mbits, jnp.float32)
    ef = e.astype(jnp.float32)
    big = mf > jnp.float32(_SQRT2)
    mf = jnp.where(big, mf * jnp.float32(0.5), mf)
    ef = jnp.where(big, ef + jnp.float32(1.0), ef)
    t = (mf - jnp.float32(1.0)) / (mf + jnp.float32(1.0))
    z = t * t
    p = jnp.float32(1.0) + z * (
        jnp.float32(1.0 / 3.0)
        + z * (jnp.float32(1.0 / 5.0) + z * jnp.float32(1.0 / 7.0))
    )
    return ef + jnp.float32(2.0 * 1.4426950408889634) * t * p


@functools.partial(
    pl.kernel,
    out_type=jax.ShapeDtypeStruct((_N_SC,), jnp.float32),
    mesh=plsc.VectorSubcoreMesh(core_axis_name="c", subcore_axis_name="s"),
    compiler_params=pltpu.CompilerParams(needs_layout_passes=False),
    scratch_types=[
        pltpu.VMEM((_BLK,), jnp.float32),
        pltpu.VMEM((_BLK,), jnp.float32),
        pltpu.VMEM((_BLK,), jnp.float32),
        pltpu.VMEM((5 * _NCOMP, 16), jnp.float32),
    ],
)
def _gmm_sc(xt_hbm, coef_hbm, out_hbm, x0_v, x1_v, out_v, coef_v):
    wid = lax.axis_index("s") * 2 + lax.axis_index("c")
    pltpu.sync_copy(coef_hbm, coef_v)
    coefs = [coef_v[i] for i in range(5 * _NCOMP)]

    for blk in range(_NBLK):
        base_in = _N_TC + (wid * _NBLK + blk) * _BLK
        pltpu.sync_copy(xt_hbm.at[0, pl.ds(base_in, _BLK)], x0_v)
        pltpu.sync_copy(xt_hbm.at[1, pl.ds(base_in, _BLK)], x1_v)

        @plsc.parallel_loop(0, _BLK // 16, unroll=4)
        def g_body(g):
            sl = pl.ds(g * 16, 16)
            x0 = x0_v[sl]
            x1 = x1_v[sl]
            sx0 = x0 * x0
            sx1 = x1 * x1
            a = []
            for i in range(_NCOMP):
                ci, b0, b1, w0, w1 = coefs[5 * i : 5 * i + 5]
                t = ci + b0 * x0
                t = t + b1 * x1
                t = t - w0 * sx0
                t = t - w1 * sx1
                a.append(t)
            m01 = jnp.maximum(a[0], a[1])
            m23 = jnp.maximum(a[2], a[3])
            m45 = jnp.maximum(a[4], a[5])
            m67 = jnp.maximum(a[6], a[7])
            m = jnp.maximum(jnp.maximum(m01, m23), jnp.maximum(m45, m67))
            s = jnp.exp(jnp.maximum(a[0] - m, jnp.float32(-40.0)))
            for i in range(1, _NCOMP):
                s = s + jnp.exp(jnp.maximum(a[i] - m, jnp.float32(-40.0)))
            out_v[sl] = m + _log2_sc(s) * jnp.float32(_LN2)

        base = (wid * _NBLK + blk) * _BLK
        pltpu.sync_copy(out_v, out_hbm.at[pl.ds(base, _BLK)])


def _gmm_tc_body(xt_ref, coef_ref, out_ref):
    for ck in range(_TC_BLK // _TC_CHUNK):
        sl = pl.ds(ck * _TC_CHUNK, _TC_CHUNK)
        x0 = xt_ref[0, sl]
        x1 = xt_ref[1, sl]
        sx0 = x0 * x0
        sx1 = x1 * x1
        a = []
        for i in range(_NCOMP):
            ci = coef_ref[i, 0]
            b0 = coef_ref[i, 1]
            b1 = coef_ref[i, 2]
            w0 = coef_ref[i, 3]
            w1 = coef_ref[i, 4]
            a.append(ci + b0 * x0 + b1 * x1 - w0 * sx0 - w1 * sx1)
        m01 = jnp.maximum(a[0], a[1])
        m23 = jnp.maximum(a[2], a[3])
        m45 = jnp.maximum(a[4], a[5])
        m67 = jnp.maximum(a[6], a[7])
        m = jnp.maximum(jnp.maximum(m01, m23), jnp.maximum(m45, m67))
        s = jnp.exp2(jnp.maximum(a[0] - m, jnp.float32(-60.0)))
        for i in range(1, _NCOMP):
            s = s + jnp.exp2(jnp.maximum(a[i] - m, jnp.float32(-60.0)))
        out_ref[sl] = (m + jnp.log2(s)) * jnp.float32(_LN2)


_gmm_tc = pl.pallas_call(
    _gmm_tc_body,
    grid=(_N_TC // _TC_BLK,),
    in_specs=[
        pl.BlockSpec((2, _TC_BLK), lambda i: (0, i)),
        pl.BlockSpec((_NCOMP, 5), lambda i: (0, 0)),
    ],
    out_specs=pl.BlockSpec((_TC_BLK,), lambda i: (i,)),
    out_shape=jax.ShapeDtypeStruct((_N,), jnp.float32),
    compiler_params=pltpu.CompilerParams(
        dimension_semantics=("arbitrary",),
    ),
)


def kernel(samples, p_s, mu_s, sigma_s):
    n, d = mu_s.shape
    var = jnp.stack([sigma_s[:, 0, 0], sigma_s[:, 1, 1]], axis=-1)
    logp = jnp.log(p_s / jnp.sum(p_s))
    w = 0.5 / var
    b = mu_s / var
    c = (
        logp
        - 0.5 * (jnp.sum(jnp.log(var), axis=-1) + d * jnp.log(2.0 * jnp.pi))
        - 0.5 * jnp.sum(mu_s * mu_s / var, axis=-1)
    )
    coef = jnp.stack([c, b[:, 0], b[:, 1], w[:, 0], w[:, 1]], axis=1)
    coef = coef.astype(jnp.float32)
    xt = samples.T
    coef_v = jnp.broadcast_to(coef[:, :, None], (n, 5, 16)).reshape(5 * n, 16)
    out_sc = _gmm_sc(xt, coef_v)
    out_tc = _gmm_tc(xt, coef * jnp.float32(1.4426950408889634))
    return lax.dynamic_update_slice(out_tc, out_sc, (_N_TC,))


# speedup vs baseline: 1.0113x; 1.0113x over previous
import functools

import jax
import jax.numpy as jnp
from jax import lax
from jax.experimental import pallas as pl
from jax.experimental.pallas import tpu as pltpu
from jax.experimental.pallas import tpu_sc as plsc

_N = 4194304
_NCOMP = 8
_NW = 32
_BLK = 8192
_N_SC = 786432
_N_TC = _N - _N_SC
_PER_W = _N_SC // _NW
_NBLK = _PER_W // _BLK

_TC_BLK = 131072
_TC_CHUNK = 4096

_LN2 = 0.6931471805599453
_SQRT2 = 1.4142135623730951


def _log2_sc(s):
    bits = lax.bitcast_convert_type(s, jnp.int32)
    e = lax.shift_right_arithmetic(bits, 23) - 127
    mbits = (bits & jnp.int32(0x007FFFFF)) | jnp.int32(0x3F800000)
    mf = lax.bitcast_convert_type(mbits, jnp.float32)
    ef = e.astype(jnp.float32)
    big = mf > jnp.float32(_SQRT2)
    mf = jnp.where(big, mf * jnp.float32(0.5), mf)
    ef = jnp.where(big, ef + jnp.float32(1.0), ef)
    t = (mf - jnp.float32(1.0)) / (mf + jnp.float32(1.0))
    z = t * t
    p = jnp.float32(1.0) + z * (
        jnp.float32(1.0 / 3.0)
        + z * (jnp.float32(1.0 / 5.0) + z * jnp.float32(1.0 / 7.0))
    )
    return ef + jnp.float32(2.0 * 1.4426950408889634) * t * p


@functools.partial(
    pl.kernel,
    out_type=jax.ShapeDtypeStruct((_N_SC,), jnp.float32),
    mesh=plsc.VectorSubcoreMesh(core_axis_name="c", subcore_axis_name="s"),
    compiler_params=pltpu.CompilerParams(needs_layout_passes=False),
    scratch_types=[
        pltpu.VMEM((_BLK,), jnp.float32),
        pltpu.VMEM((_BLK,), jnp.float32),
        pltpu.VMEM((_BLK,), jnp.float32),
        pltpu.VMEM((5 * _NCOMP, 16), jnp.float32),
    ],
)
def _gmm_sc(xt_hbm, coef_hbm, out_hbm, x0_v, x1_v, out_v, coef_v):
    wid = lax.axis_index("s") * 2 + lax.axis_index("c")
    pltpu.sync_copy(coef_hbm, coef_v)
    coefs = [coef_v[i] for i in range(5 * _NCOMP)]

    for blk in range(_NBLK):
        base_in = _N_TC + (wid * _NBLK + blk) * _BLK
        pltpu.sync_copy(xt_hbm.at[0, pl.ds(base_in, _BLK)], x0_v)
        pltpu.sync_copy(xt_hbm.at[1, pl.ds(base_in, _BLK)], x1_v)

        @plsc.parallel_loop(0, _BLK // 16, unroll=4)
        def g_body(g):
            sl = pl.ds(g * 16, 16)
            x0 = x0_v[sl]
            x1 = x1_v[sl]
            sx0 = x0 * x0
            sx1 = x1 * x1
            a = []
            for i in range(_NCOMP):
                ci, b0, b1, w0, w1 = coefs[5 * i : 5 * i + 5]
                t = ci + b0 * x0
                t = t + b1 * x1
                t = t - w0 * sx0
                t = t - w1 * sx1
                a.append(t)
            m01 = jnp.maximum(a[0], a[1])
            m23 = jnp.maximum(a[2], a[3])
            m45 = jnp.maximum(a[4], a[5])
            m67 = jnp.maximum(a[6], a[7])
            m = jnp.maximum(jnp.maximum(m01, m23), jnp.maximum(m45, m67))
            s = jnp.exp(jnp.maximum(a[0] - m, jnp.float32(-40.0)))
            for i in range(1, _NCOMP):
                s = s + jnp.exp(jnp.maximum(a[i] - m, jnp.float32(-40.0)))
            out_v[sl] = m + _log2_sc(s) * jnp.float32(_LN2)

        base = (wid * _NBLK + blk) * _BLK
        pltpu.sync_copy(out_v, out_hbm.at[pl.ds(base, _BLK)])


def _gmm_tc_body(xt_ref, coef_ref, out_ref):
    for ck in range(_TC_BLK // _TC_CHUNK):
        sl = pl.ds(ck * _TC_CHUNK, _TC_CHUNK)
        x0 = xt_ref[0, sl]
        x1 = xt_ref[1, sl]
        sx0 = x0 * x0
        sx1 = x1 * x1
        a = []
        for i in range(_NCOMP):
            ci = coef_ref[i, 0]
            b0 = coef_ref[i, 1]
            b1 = coef_ref[i, 2]
            w0 = coef_ref[i, 3]
            w1 = coef_ref[i, 4]
            a.append(ci + b0 * x0 + b1 * x1 - w0 * sx0 - w1 * sx1)
        m01 = jnp.maximum(a[0], a[1])
        m23 = jnp.maximum(a[2], a[3])
        m45 = jnp.maximum(a[4], a[5])
        m67 = jnp.maximum(a[6], a[7])
        m = jnp.maximum(jnp.maximum(m01, m23), jnp.maximum(m45, m67))
        s = jnp.exp2(jnp.maximum(a[0] - m, jnp.float32(-60.0)))
        for i in range(1, _NCOMP):
            s = s + jnp.exp2(jnp.maximum(a[i] - m, jnp.float32(-60.0)))
        out_ref[sl] = (m + jnp.log2(s)) * jnp.float32(_LN2)


_gmm_tc = pl.pallas_call(
    _gmm_tc_body,
    grid=(_N_TC // _TC_BLK,),
    in_specs=[
        pl.BlockSpec((2, _TC_BLK), lambda i: (0, i)),
        pl.BlockSpec((_NCOMP, 5), lambda i: (0, 0)),
    ],
    out_specs=pl.BlockSpec((_TC_BLK,), lambda i: (i,)),
    out_shape=jax.ShapeDtypeStruct((_N,), jnp.float32),
    compiler_params=pltpu.CompilerParams(
        dimension_semantics=("arbitrary",),
    ),
)


def kernel(samples, p_s, mu_s, sigma_s):
    n, d = mu_s.shape
    var = jnp.diagonal(sigma_s, axis1=-2, axis2=-1)
    logp = jnp.log(p_s / jnp.sum(p_s))
    w = 0.5 / var
    b = mu_s / var
    c = (
        logp
        - 0.5 * (jnp.sum(jnp.log(var), axis=-1) + d * jnp.log(2.0 * jnp.pi))
        - 0.5 * jnp.sum(mu_s * mu_s / var, axis=-1)
    )
    coef = jnp.stack([c, b[:, 0], b[:, 1], w[:, 0], w[:, 1]], axis=1)
    coef = coef.astype(jnp.float32)
    xt = samples.T
    coef_v = jnp.broadcast_to(coef[:, :, None], (n, 5, 16)).reshape(5 * n, 16)
    out_sc = _gmm_sc(xt, coef_v)
    out_tc = _gmm_tc(xt, coef * jnp.float32(1.4426950408889634))
    return lax.dynamic_update_slice(out_tc, out_sc, (_N_TC,))
